# baseline (speedup 1.0000x reference)
import functools

import jax
import jax.numpy as jnp
from jax import lax
from jax.experimental import pallas as pl
from jax.experimental.pallas import tpu as pltpu
from jax.experimental.pallas import tpu_sc as plsc

_VOCAB = 1000
_D = 32
_N = 128
_BATCH = 16384
_HIST = 200
_LANES = 16
_NC = 2
_NS = 16
_NW = _NC * _NS
_ROWS_PER_W = _BATCH // _NW
_GROUPS = _ROWS_PER_W // _LANES
_BM = 1024


def _hist_body(bos_hbm, hist_hbm, bos_v, hist_v):
  wid = lax.axis_index("s") * _NC + lax.axis_index("c")
  base = wid * _ROWS_PER_W

  iota = lax.iota(jnp.int32, _LANES)
  col0 = iota * _HIST
  rowoff = iota * _VOCAB
  ones = jnp.full((_LANES,), 1.0, jnp.float32)
  neg_ones = jnp.full((_LANES,), -1.0, jnp.float32)
  zf = jnp.zeros((_LANES,), jnp.float32)

  for k in range(_LANES * _VOCAB // _LANES):
    hist_v[pl.ds(k * _LANES, _LANES)] = zf

  def group_body(g, _):
    row0 = base + g * _LANES
    pltpu.sync_copy(bos_hbm.at[pl.ds(row0 * _HIST, _LANES * _HIST)], bos_v)
    for l in range(_HIST):
      tokc = plsc.load_gather(bos_v, [col0 + l])
      plsc.addupdate_scatter(hist_v, [tokc + rowoff], ones)
    pltpu.sync_copy(hist_v, hist_hbm.at[pl.ds(row0 * _VOCAB,
                                              _LANES * _VOCAB)])
    for l in range(_HIST):
      tokc = plsc.load_gather(bos_v, [col0 + l])
      plsc.addupdate_scatter(hist_v, [tokc + rowoff], neg_ones)
    return 0

  lax.fori_loop(0, _GROUPS, group_body, 0)


def _mm_body(hist_ref, tab_ref, out_ref):
  h = hist_ref[...].astype(jnp.bfloat16)
  r = jnp.dot(h, tab_ref[...], preferred_element_type=jnp.float32)
  out_ref[...] = r[:, :_D] / r[:, _D:_D + 1]


@functools.partial(jax.jit, donate_argnums=())
def _run(bos_flat, tab_aug):
  mesh = plsc.VectorSubcoreMesh(core_axis_name="c", subcore_axis_name="s")
  hist_k = pl.kernel(
      _hist_body,
      out_type=jax.ShapeDtypeStruct((_BATCH * _VOCAB,), jnp.float32),
      mesh=mesh,
      scratch_types=[
          pltpu.VMEM((_LANES * _HIST,), jnp.int32),
          pltpu.VMEM((_LANES * _VOCAB,), jnp.float32),
      ],
      compiler_params=pltpu.CompilerParams(needs_layout_passes=False),
  )
  hist = hist_k(bos_flat).reshape(_BATCH, _VOCAB)

  mm = pl.pallas_call(
      _mm_body,
      grid=(_BATCH // _BM,),
      in_specs=[
          pl.BlockSpec((_BM, _VOCAB), lambda i: (i, 0)),
          pl.BlockSpec((_VOCAB, _N), lambda i: (0, 0)),
      ],
      out_specs=pl.BlockSpec((_BM, _D), lambda i: (i, 0)),
      out_shape=jax.ShapeDtypeStruct((_BATCH, _D), jnp.float32),
  )
  return mm(hist, tab_aug)


def kernel(bos, table):
  tab_eff = table.at[0].set(0.0)
  ones_col = jnp.ones((_VOCAB, 1), jnp.float32).at[0, 0].set(0.0)
  pad = jnp.zeros((_VOCAB, _N - _D - 1), jnp.float32)
  tab_aug = jnp.concatenate([tab_eff, ones_col, pad], 1).astype(jnp.bfloat16)
  return _run(bos.reshape(-1), tab_aug)


# speedup vs baseline: 1.6451x; 1.6451x over previous
import functools

import jax
import jax.numpy as jnp
from jax import lax
from jax.experimental import pallas as pl
from jax.experimental.pallas import tpu as pltpu
from jax.experimental.pallas import tpu_sc as plsc

_VOCAB = 1000
_D = 32
_N = 128
_BATCH = 16384
_HIST = 200
_LANES = 16
_NC = 2
_NS = 16
_NW = _NC * _NS
_ROWS_PER_W = _BATCH // _NW
_GROUPS = _ROWS_PER_W // _LANES
_GBOS = _LANES * _HIST
_GHIST = _LANES * _VOCAB
_BM = 512


def _hist_body(bos_hbm, hist_hbm, bos_a, bos_b, hist_a, hist_b,
               sem_ba, sem_bb, sem_ha, sem_hb):
  wid = lax.axis_index("s") * _NC + lax.axis_index("c")
  base = wid * _ROWS_PER_W

  iota = lax.iota(jnp.int32, _LANES)
  col0 = iota * _HIST
  rowoff = iota * _VOCAB
  ones = jnp.full((_LANES,), 1.0, jnp.float32)
  zf = jnp.zeros((_LANES,), jnp.float32)

  bos_bufs = (bos_a, bos_b)
  hist_bufs = (hist_a, hist_b)
  bos_sems = (sem_ba, sem_bb)
  hist_sems = (sem_ha, sem_hb)

  def bos_src(g):
    return bos_hbm.at[pl.ds((base + g * _LANES) * _HIST, _GBOS)]

  def hist_dst(g):
    return hist_hbm.at[pl.ds((base + g * _LANES) * _VOCAB, _GHIST)]

  pltpu.async_copy(bos_src(0), bos_a, sem_ba)

  def pair_body(k, _):
    for p in range(2):
      g = 2 * k + p
      bos_v, hist_v = bos_bufs[p], hist_bufs[p]
      bsem, hsem = bos_sems[p], hist_sems[p]

      @pl.when(g >= 2)
      def _wait_hist():
        pltpu.make_async_copy(hist_v, hist_dst(0), hsem).wait()

      @plsc.parallel_loop(0, _VOCAB, 1, unroll=8)
      def _zero(i):
        hist_v[pl.ds(i * _LANES, _LANES)] = zf

      pltpu.make_async_copy(bos_src(0), bos_v, bsem).wait()

      nb = (p + 1) % 2

      @pl.when(g + 1 < _GROUPS)
      def _prefetch():
        pltpu.async_copy(bos_src(g + 1), bos_bufs[nb], bos_sems[nb])

      @plsc.parallel_loop(0, _HIST, 1, unroll=8)
      def _scatter(l):
        tokc = plsc.load_gather(bos_v, [col0 + l])
        plsc.addupdate_scatter(hist_v, [tokc + rowoff], ones)

      pltpu.async_copy(hist_v, hist_dst(g), hsem)
    return 0

  lax.fori_loop(0, _GROUPS // 2, pair_body, 0)

  pltpu.make_async_copy(hist_a, hist_dst(0), sem_ha).wait()
  pltpu.make_async_copy(hist_b, hist_dst(0), sem_hb).wait()


def _mm_body(hist_ref, tab_ref, out_ref):
  h = hist_ref[...].astype(jnp.bfloat16)
  r = jnp.dot(h, tab_ref[...], preferred_element_type=jnp.float32)
  out_ref[...] = r[:, :_D] / r[:, _D:_D + 1]


@functools.partial(jax.jit, donate_argnums=())
def _run(bos_flat, tab_aug):
  mesh = plsc.VectorSubcoreMesh(core_axis_name="c", subcore_axis_name="s")
  hist_k = pl.kernel(
      _hist_body,
      out_type=jax.ShapeDtypeStruct((_BATCH * _VOCAB,), jnp.float32),
      mesh=mesh,
      scratch_types=[
          pltpu.VMEM((_GBOS,), jnp.int32),
          pltpu.VMEM((_GBOS,), jnp.int32),
          pltpu.VMEM((_GHIST,), jnp.float32),
          pltpu.VMEM((_GHIST,), jnp.float32),
          pltpu.SemaphoreType.DMA,
          pltpu.SemaphoreType.DMA,
          pltpu.SemaphoreType.DMA,
          pltpu.SemaphoreType.DMA,
      ],
      compiler_params=pltpu.CompilerParams(needs_layout_passes=False),
  )
  hist = hist_k(bos_flat).reshape(_BATCH, _VOCAB)

  mm = pl.pallas_call(
      _mm_body,
      grid=(_BATCH // _BM,),
      in_specs=[
          pl.BlockSpec((_BM, _VOCAB), lambda i: (i, 0)),
          pl.BlockSpec((_VOCAB, _N), lambda i: (0, 0)),
      ],
      out_specs=pl.BlockSpec((_BM, _D), lambda i: (i, 0)),
      out_shape=jax.ShapeDtypeStruct((_BATCH, _D), jnp.float32),
  )
  return mm(hist, tab_aug)


def kernel(bos, table):
  tab_eff = table.at[0].set(0.0)
  ones_col = jnp.ones((_VOCAB, 1), jnp.float32).at[0, 0].set(0.0)
  pad = jnp.zeros((_VOCAB, _N - _D - 1), jnp.float32)
  tab_aug = jnp.concatenate([tab_eff, ones_col, pad], 1).astype(jnp.bfloat16)
  return _run(bos.reshape(-1), tab_aug)


# speedup vs baseline: 1.6872x; 1.0256x over previous
import functools

import jax
import jax.numpy as jnp
from jax import lax
from jax.experimental import pallas as pl
from jax.experimental.pallas import tpu as pltpu
from jax.experimental.pallas import tpu_sc as plsc

_VOCAB = 1000
_VPAD = 1024
_D = 32
_N = 128
_BATCH = 16384
_HIST = 200
_LANES = 16
_NC = 2
_NS = 16
_NW = _NC * _NS
_ROWS_PER_W = _BATCH // _NW
_GROUPS = _ROWS_PER_W // _LANES
_GBOS = _LANES * _HIST
_GHIST = _LANES * _VPAD
_BM = 512


def _hist_body(bos_hbm, hist_hbm, bos_a, bos_b, hist_a, hist_b,
               sem_ba, sem_bb, sem_ha, sem_hb):
  wid = lax.axis_index("s") * _NC + lax.axis_index("c")
  base = wid * _ROWS_PER_W

  iota = lax.iota(jnp.int32, _LANES)
  col0 = iota * _HIST
  rowoff = iota * _VPAD
  ones = jnp.full((_LANES,), 1.0, jnp.float32)
  zf = jnp.zeros((_LANES,), jnp.float32)

  bos_bufs = (bos_a, bos_b)
  hist_bufs = (hist_a, hist_b)
  bos_sems = (sem_ba, sem_bb)
  hist_sems = (sem_ha, sem_hb)

  def bos_src(g):
    return bos_hbm.at[pl.ds((base + g * _LANES) * _HIST, _GBOS)]

  def hist_dst(g):
    return hist_hbm.at[pl.ds((base + g * _LANES) * _VPAD, _GHIST)]

  pltpu.async_copy(bos_src(0), bos_a, sem_ba)

  def pair_body(k, _):
    for p in range(2):
      g = 2 * k + p
      bos_v, hist_v = bos_bufs[p], hist_bufs[p]
      bsem, hsem = bos_sems[p], hist_sems[p]

      @pl.when(g >= 2)
      def _wait_hist():
        pltpu.make_async_copy(hist_v, hist_dst(0), hsem).wait()

      @plsc.parallel_loop(0, _VPAD, 1, unroll=8)
      def _zero(i):
        hist_v[pl.ds(i * _LANES, _LANES)] = zf

      pltpu.make_async_copy(bos_src(0), bos_v, bsem).wait()

      nb = (p + 1) % 2

      @pl.when(g + 1 < _GROUPS)
      def _prefetch():
        pltpu.async_copy(bos_src(g + 1), bos_bufs[nb], bos_sems[nb])

      @plsc.parallel_loop(0, _HIST, 1, unroll=8)
      def _scatter(l):
        tokc = plsc.load_gather(bos_v, [col0 + l])
        plsc.addupdate_scatter(hist_v, [tokc + rowoff], ones)

      pltpu.async_copy(hist_v, hist_dst(g), hsem)
    return 0

  lax.fori_loop(0, _GROUPS // 2, pair_body, 0)

  pltpu.make_async_copy(hist_a, hist_dst(0), sem_ha).wait()
  pltpu.make_async_copy(hist_b, hist_dst(0), sem_hb).wait()


def _mm_body(hist_ref, tab_ref, out_ref):
  h = hist_ref[...].astype(jnp.bfloat16)
  r = jnp.dot(h, tab_ref[...], preferred_element_type=jnp.float32)
  out_ref[...] = r[:, :_D] / r[:, _D:_D + 1]


@functools.partial(jax.jit, donate_argnums=())
def _run(bos_flat, tab_aug):
  mesh = plsc.VectorSubcoreMesh(core_axis_name="c", subcore_axis_name="s")
  hist_k = pl.kernel(
      _hist_body,
      out_type=jax.ShapeDtypeStruct((_BATCH * _VPAD,), jnp.float32),
      mesh=mesh,
      scratch_types=[
          pltpu.VMEM((_GBOS,), jnp.int32),
          pltpu.VMEM((_GBOS,), jnp.int32),
          pltpu.VMEM((_GHIST,), jnp.float32),
          pltpu.VMEM((_GHIST,), jnp.float32),
          pltpu.SemaphoreType.DMA,
          pltpu.SemaphoreType.DMA,
          pltpu.SemaphoreType.DMA,
          pltpu.SemaphoreType.DMA,
      ],
      compiler_params=pltpu.CompilerParams(needs_layout_passes=False),
  )
  hist = hist_k(bos_flat).reshape(_BATCH, _VPAD)

  mm = pl.pallas_call(
      _mm_body,
      grid=(_BATCH // _BM,),
      in_specs=[
          pl.BlockSpec((_BM, _VPAD), lambda i: (i, 0)),
          pl.BlockSpec((_VPAD, _N), lambda i: (0, 0)),
      ],
      out_specs=pl.BlockSpec((_BM, _D), lambda i: (i, 0)),
      out_shape=jax.ShapeDtypeStruct((_BATCH, _D), jnp.float32),
  )
  return mm(hist, tab_aug)


def kernel(bos, table):
  tab_eff = table.at[0].set(0.0)
  ones_col = jnp.ones((_VOCAB, 1), jnp.float32).at[0, 0].set(0.0)
  pad = jnp.zeros((_VOCAB, _N - _D - 1), jnp.float32)
  tab_aug = jnp.concatenate([tab_eff, ones_col, pad], 1).astype(jnp.bfloat16)
  tab_aug = jnp.pad(tab_aug, ((0, _VPAD - _VOCAB), (0, 0)))
  return _run(bos.reshape(-1), tab_aug)


# speedup vs baseline: 2.3715x; 1.4056x over previous
import functools

import jax
import jax.numpy as jnp
from jax import lax
from jax.experimental import pallas as pl
from jax.experimental.pallas import tpu as pltpu
from jax.experimental.pallas import tpu_sc as plsc

_VOCAB = 1000
_VPAD = 1024
_D = 32
_N = 128
_BATCH = 16384
_HIST = 200
_LANES = 16
_NC = 2
_NS = 16
_NW = _NC * _NS
_ROWS_PER_W = _BATCH // _NW
_GROUPS = _ROWS_PER_W // _LANES
_GBOS = _LANES * _HIST
_GHIST = _LANES * _VPAD
_BM = 512


def _hist_body(bos_hbm, hist_hbm, bos_a, bos_b, hist_a, hist_b,
               sem_ba, sem_bb, sem_ha, sem_hb):
  wid = lax.axis_index("s") * _NC + lax.axis_index("c")
  base = wid * _ROWS_PER_W

  iota = lax.iota(jnp.int32, _LANES)
  lane_iota = iota
  zeros_i = jnp.zeros((_LANES,), jnp.int32)
  rowoff = iota * _VPAD
  ones = jnp.full((_LANES,), 1.0, jnp.float32)
  zf = jnp.zeros((_LANES,), jnp.float32)

  bos_bufs = (bos_a, bos_b)
  hist_bufs = (hist_a, hist_b)
  bos_sems = (sem_ba, sem_bb)
  hist_sems = (sem_ha, sem_hb)

  def bos_src(g):
    return bos_hbm.at[pl.ds(base + g * _LANES, _LANES), :]

  def hist_dst(g):
    return hist_hbm.at[pl.ds((base + g * _LANES) * _VPAD, _GHIST)]

  pltpu.async_copy(bos_src(0), bos_a, sem_ba)

  def pair_body(k, _):
    for p in range(2):
      g = 2 * k + p
      bos_v, hist_v = bos_bufs[p], hist_bufs[p]
      bsem, hsem = bos_sems[p], hist_sems[p]

      @pl.when(g >= 2)
      def _wait_hist():
        pltpu.make_async_copy(hist_v, hist_dst(0), hsem).wait()

      @plsc.parallel_loop(0, _VPAD, 1, unroll=8)
      def _zero(i):
        hist_v[pl.ds(i * _LANES, _LANES)] = zf

      pltpu.make_async_copy(bos_src(0), bos_v, bsem).wait()

      nb = (p + 1) % 2

      @pl.when(g + 1 < _GROUPS)
      def _prefetch():
        pltpu.async_copy(bos_src(g + 1), bos_bufs[nb], bos_sems[nb])

      @plsc.parallel_loop(0, _HIST, 1, unroll=8)
      def _scatter(l):
        tokc = plsc.load_gather(bos_v, [lane_iota, zeros_i + l])
        plsc.addupdate_scatter(hist_v, [tokc + rowoff], ones)

      pltpu.async_copy(hist_v, hist_dst(g), hsem)
    return 0

  lax.fori_loop(0, _GROUPS // 2, pair_body, 0)

  pltpu.make_async_copy(hist_a, hist_dst(0), sem_ha).wait()
  pltpu.make_async_copy(hist_b, hist_dst(0), sem_hb).wait()


def _mm_body(hist_ref, tab_ref, out_ref):
  h = hist_ref[...].reshape(_BM, _VPAD).astype(jnp.bfloat16)
  r = jnp.dot(h, tab_ref[...], preferred_element_type=jnp.float32)
  out_ref[...] = r[:, :_D] / r[:, _D:_D + 1]


@functools.partial(jax.jit, donate_argnums=())
def _run(bos, tab_aug):
  mesh = plsc.VectorSubcoreMesh(core_axis_name="c", subcore_axis_name="s")
  hist_k = pl.kernel(
      _hist_body,
      out_type=jax.ShapeDtypeStruct((_BATCH * _VPAD,), jnp.float32),
      mesh=mesh,
      scratch_types=[
          pltpu.VMEM((_LANES, _HIST), jnp.int32),
          pltpu.VMEM((_LANES, _HIST), jnp.int32),
          pltpu.VMEM((_GHIST,), jnp.float32),
          pltpu.VMEM((_GHIST,), jnp.float32),
          pltpu.SemaphoreType.DMA,
          pltpu.SemaphoreType.DMA,
          pltpu.SemaphoreType.DMA,
          pltpu.SemaphoreType.DMA,
      ],
      compiler_params=pltpu.CompilerParams(needs_layout_passes=False),
  )
  hist = hist_k(bos)

  mm = pl.pallas_call(
      _mm_body,
      grid=(_BATCH // _BM,),
      in_specs=[
          pl.BlockSpec((_BM * _VPAD,), lambda i: (i,)),
          pl.BlockSpec((_VPAD, _N), lambda i: (0, 0)),
      ],
      out_specs=pl.BlockSpec((_BM, _D), lambda i: (i, 0)),
      out_shape=jax.ShapeDtypeStruct((_BATCH, _D), jnp.float32),
  )
  return mm(hist, tab_aug)


def kernel(bos, table):
  tab_eff = table.at[0].set(0.0)
  ones_col = jnp.ones((_VOCAB, 1), jnp.float32).at[0, 0].set(0.0)
  pad = jnp.zeros((_VOCAB, _N - _D - 1), jnp.float32)
  tab_aug = jnp.concatenate([tab_eff, ones_col, pad], 1).astype(jnp.bfloat16)
  tab_aug = jnp.pad(tab_aug, ((0, _VPAD - _VOCAB), (0, 0)))
  return _run(bos, tab_aug)


# speedup vs baseline: 2.4624x; 1.0383x over previous
import functools

import jax
import jax.numpy as jnp
from jax import lax
from jax.experimental import pallas as pl
from jax.experimental.pallas import tpu as pltpu
from jax.experimental.pallas import tpu_sc as plsc

_VOCAB = 1000
_VPAD = 1024
_D = 32
_N = 128
_BATCH = 16384
_HIST = 200
_LANES = 16
_NC = 2
_NS = 16
_NW = _NC * _NS
_ROWS_PER_W = _BATCH // _NW
_CROWS = 128
_NCHUNK = _ROWS_PER_W // _CROWS
_GPC = _CROWS // _LANES
_GHIST = _LANES * _VPAD
_BM = 1024


def _hist_body(bos_hbm, hist_hbm, bos_a, bos_b, hist_a, hist_b,
               sem_ba, sem_bb, sem_ha, sem_hb):
  wid = lax.axis_index("s") * _NC + lax.axis_index("c")
  base = wid * _ROWS_PER_W

  iota = lax.iota(jnp.int32, _LANES)
  rowoff = iota * _VPAD
  zeros_i = jnp.zeros((_LANES,), jnp.int32)
  ones = jnp.full((_LANES,), 1.0, jnp.float32)
  zf = jnp.zeros((_LANES,), jnp.float32)

  bos_bufs = (bos_a, bos_b)
  hist_bufs = (hist_a, hist_b)
  bos_sems = (sem_ba, sem_bb)
  hist_sems = (sem_ha, sem_hb)

  def bos_src(c):
    return bos_hbm.at[pl.ds(base + c * _CROWS, _CROWS), :]

  def hist_dst(gg):
    return hist_hbm.at[pl.ds((base + gg * _LANES) * _VPAD, _GHIST)]

  pltpu.async_copy(bos_src(0), bos_a, sem_ba)

  for c in range(_NCHUNK):
    s = c % 2
    bos_v = bos_bufs[s]
    pltpu.make_async_copy(bos_src(0), bos_v, bos_sems[s]).wait()
    if c + 1 < _NCHUNK:
      pltpu.async_copy(bos_src(c + 1), bos_bufs[1 - s], bos_sems[1 - s])

    for lg in range(_GPC):
      gg = c * _GPC + lg
      p = gg % 2
      hist_v = hist_bufs[p]
      hsem = hist_sems[p]
      if gg >= 2:
        pltpu.make_async_copy(hist_v, hist_dst(0), hsem).wait()

      @plsc.parallel_loop(0, _VPAD, 1, unroll=8)
      def _zero(i):
        hist_v[pl.ds(i * _LANES, _LANES)] = zf

      lrow = iota + lg * _LANES

      @plsc.parallel_loop(0, _HIST, 1, unroll=8)
      def _scatter(l):
        tokc = plsc.load_gather(bos_v, [lrow, zeros_i + l])
        plsc.addupdate_scatter(hist_v, [tokc + rowoff], ones)

      pltpu.async_copy(hist_v, hist_dst(gg), hsem)

  pltpu.make_async_copy(hist_a, hist_dst(0), sem_ha).wait()
  pltpu.make_async_copy(hist_b, hist_dst(0), sem_hb).wait()


def _mm_body(hist_ref, tab_ref, out_ref):
  h = hist_ref[...].reshape(_BM, _VPAD).astype(jnp.bfloat16)
  r = jnp.dot(h, tab_ref[...], preferred_element_type=jnp.float32)
  out_ref[...] = r[:, :_D] / r[:, _D:_D + 1]


@functools.partial(jax.jit, donate_argnums=())
def _run(bos, tab_aug):
  mesh = plsc.VectorSubcoreMesh(core_axis_name="c", subcore_axis_name="s")
  hist_k = pl.kernel(
      _hist_body,
      out_type=jax.ShapeDtypeStruct((_BATCH * _VPAD,), jnp.float32),
      mesh=mesh,
      scratch_types=[
          pltpu.VMEM((_CROWS, _HIST), jnp.int32),
          pltpu.VMEM((_CROWS, _HIST), jnp.int32),
          pltpu.VMEM((_GHIST,), jnp.float32),
          pltpu.VMEM((_GHIST,), jnp.float32),
          pltpu.SemaphoreType.DMA,
          pltpu.SemaphoreType.DMA,
          pltpu.SemaphoreType.DMA,
          pltpu.SemaphoreType.DMA,
      ],
      compiler_params=pltpu.CompilerParams(needs_layout_passes=False),
  )
  hist = hist_k(bos)

  mm = pl.pallas_call(
      _mm_body,
      grid=(_BATCH // _BM,),
      in_specs=[
          pl.BlockSpec((_BM * _VPAD,), lambda i: (i,)),
          pl.BlockSpec((_VPAD, _N), lambda i: (0, 0)),
      ],
      out_specs=pl.BlockSpec((_BM, _D), lambda i: (i, 0)),
      out_shape=jax.ShapeDtypeStruct((_BATCH, _D), jnp.float32),
  )
  return mm(hist, tab_aug)


def kernel(bos, table):
  tab_eff = table.at[0].set(0.0)
  ones_col = jnp.ones((_VOCAB, 1), jnp.float32).at[0, 0].set(0.0)
  pad = jnp.zeros((_VOCAB, _N - _D - 1), jnp.float32)
  tab_aug = jnp.concatenate([tab_eff, ones_col, pad], 1).astype(jnp.bfloat16)
  tab_aug = jnp.pad(tab_aug, ((0, _VPAD - _VOCAB), (0, 0)))
  return _run(bos, tab_aug)


# speedup vs baseline: 2.6235x; 1.0654x over previous
import functools

import jax
import jax.numpy as jnp
from jax import lax
from jax.experimental import pallas as pl
from jax.experimental.pallas import tpu as pltpu
from jax.experimental.pallas import tpu_sc as plsc

_VOCAB = 1000
_VPAD = 1024
_D = 32
_N = 128
_BATCH = 16384
_HIST = 200
_LANES = 16
_NC = 2
_NS = 16
_NW = _NC * _NS
_NCHB = 4
_CB = _BATCH // _NCHB
_ROWS_PER_W = _CB // _NW
_GROUPS = _ROWS_PER_W // _LANES
_GBOS = _LANES * _HIST
_GHIST = _LANES * _VPAD
_BM = 1024


def _hist_body(bos_hbm, hist_hbm, bos_a, bos_b, hist_a, hist_b,
               sem_ba, sem_bb, sem_ha, sem_hb):
  wid = lax.axis_index("s") * _NC + lax.axis_index("c")
  base = wid * _ROWS_PER_W

  iota = lax.iota(jnp.int32, _LANES)
  col0 = iota * _HIST
  rowoff = iota * _VPAD
  ones = jnp.full((_LANES,), 1.0, jnp.float32)
  zf = jnp.zeros((_LANES,), jnp.float32)

  bos_bufs = (bos_a, bos_b)
  hist_bufs = (hist_a, hist_b)
  bos_sems = (sem_ba, sem_bb)
  hist_sems = (sem_ha, sem_hb)

  def bos_src(g):
    return bos_hbm.at[pl.ds((base + g * _LANES) * _HIST, _GBOS)]

  def hist_dst(g):
    return hist_hbm.at[pl.ds((base + g * _LANES) * _VPAD, _GHIST)]

  pltpu.async_copy(bos_src(0), bos_a, sem_ba)

  def pair_body(k, _):
    for p in range(2):
      g = 2 * k + p
      bos_v, hist_v = bos_bufs[p], hist_bufs[p]
      bsem, hsem = bos_sems[p], hist_sems[p]

      @pl.when(g >= 2)
      def _wait_hist():
        pltpu.make_async_copy(hist_v, hist_dst(0), hsem).wait()

      @plsc.parallel_loop(0, _VPAD, 1, unroll=8)
      def _zero(i):
        hist_v[pl.ds(i * _LANES, _LANES)] = zf

      pltpu.make_async_copy(bos_src(0), bos_v, bsem).wait()

      nb = (p + 1) % 2

      @pl.when(g + 1 < _GROUPS)
      def _prefetch():
        pltpu.async_copy(bos_src(g + 1), bos_bufs[nb], bos_sems[nb])

      @plsc.parallel_loop(0, _HIST, 1, unroll=8)
      def _scatter(l):
        tokc = plsc.load_gather(bos_v, [col0 + l])
        plsc.addupdate_scatter(hist_v, [tokc + rowoff], ones)

      pltpu.async_copy(hist_v, hist_dst(g), hsem)
    return 0

  lax.fori_loop(0, _GROUPS // 2, pair_body, 0)

  pltpu.make_async_copy(hist_a, hist_dst(0), sem_ha).wait()
  pltpu.make_async_copy(hist_b, hist_dst(0), sem_hb).wait()


def _mm_body(hist_ref, tab_ref, out_ref):
  h = hist_ref[...].reshape(_BM, _VPAD).astype(jnp.bfloat16)
  r = jnp.dot(h, tab_ref[...], preferred_element_type=jnp.float32)
  out_ref[...] = r[:, :_D] / r[:, _D:_D + 1]


@functools.partial(jax.jit, donate_argnums=())
def _run(bos_flat, tab_aug):
  mesh = plsc.VectorSubcoreMesh(core_axis_name="c", subcore_axis_name="s")
  hist_k = pl.kernel(
      _hist_body,
      out_type=jax.ShapeDtypeStruct((_CB * _VPAD,), jnp.float32),
      mesh=mesh,
      scratch_types=[
          pltpu.VMEM((_GBOS,), jnp.int32),
          pltpu.VMEM((_GBOS,), jnp.int32),
          pltpu.VMEM((_GHIST,), jnp.float32),
          pltpu.VMEM((_GHIST,), jnp.float32),
          pltpu.SemaphoreType.DMA,
          pltpu.SemaphoreType.DMA,
          pltpu.SemaphoreType.DMA,
          pltpu.SemaphoreType.DMA,
      ],
      compiler_params=pltpu.CompilerParams(needs_layout_passes=False),
  )
  mm = pl.pallas_call(
      _mm_body,
      grid=(_CB // _BM,),
      in_specs=[
          pl.BlockSpec((_BM * _VPAD,), lambda i: (i,)),
          pl.BlockSpec((_VPAD, _N), lambda i: (0, 0)),
      ],
      out_specs=pl.BlockSpec((_BM, _D), lambda i: (i, 0)),
      out_shape=jax.ShapeDtypeStruct((_CB, _D), jnp.float32),
  )
  outs = []
  for i in range(_NCHB):
    flat_i = lax.slice_in_dim(bos_flat, i * _CB * _HIST,
                              (i + 1) * _CB * _HIST)
    outs.append(mm(hist_k(flat_i), tab_aug))
  return jnp.concatenate(outs, 0)


def kernel(bos, table):
  tab_eff = table.at[0].set(0.0)
  ones_col = jnp.ones((_VOCAB, 1), jnp.float32).at[0, 0].set(0.0)
  pad = jnp.zeros((_VOCAB, _N - _D - 1), jnp.float32)
  tab_aug = jnp.concatenate([tab_eff, ones_col, pad], 1).astype(jnp.bfloat16)
  tab_aug = jnp.pad(tab_aug, ((0, _VPAD - _VOCAB), (0, 0)))
  return _run(bos.reshape(-1), tab_aug)


# speedup vs baseline: 2.7231x; 1.0380x over previous
import functools

import jax
import jax.numpy as jnp
from jax import lax
from jax.experimental import pallas as pl
from jax.experimental.pallas import tpu as pltpu
from jax.experimental.pallas import tpu_sc as plsc

_VOCAB = 1000
_VPAD = 1024
_D = 32
_N = 128
_BATCH = 16384
_HIST = 200
_LANES = 16
_NC = 2
_NS = 16
_NW = _NC * _NS
_NCHB = 4
_CB = _BATCH // _NCHB
_ROWS_PER_W = _CB // _NW
_GROUPS = _ROWS_PER_W // _LANES
_GBOS = _LANES * _HIST
_GHIST = _LANES * _VPAD
_BM = 1024


def _hist_body(bos_hbm, hist_hbm, bos_a, bos_b, hist_a, hist_b,
               sem_ba, sem_bb, sem_ha, sem_hb):
  wid = lax.axis_index("s") * _NC + lax.axis_index("c")
  base = wid * _ROWS_PER_W

  iota = lax.iota(jnp.int32, _LANES)
  col0 = iota * _HIST
  rowoff = iota * _VPAD
  ones = jnp.full((_LANES,), 1.0, jnp.float32)
  zf = jnp.zeros((_LANES,), jnp.float32)

  bos_bufs = (bos_a, bos_b)
  hist_bufs = (hist_a, hist_b)
  bos_sems = (sem_ba, sem_bb)
  hist_sems = (sem_ha, sem_hb)

  def bos_src(g):
    return bos_hbm.at[pl.ds((base + g * _LANES) * _HIST, _GBOS)]

  def hist_dst(g):
    return hist_hbm.at[pl.ds((base + g * _LANES) * _VPAD, _GHIST)]

  pltpu.async_copy(bos_src(0), bos_a, sem_ba)

  def pair_body(k, _):
    for p in range(2):
      g = 2 * k + p
      bos_v, hist_v = bos_bufs[p], hist_bufs[p]
      bsem, hsem = bos_sems[p], hist_sems[p]

      @pl.when(g >= 2)
      def _wait_hist():
        pltpu.make_async_copy(hist_v, hist_dst(0), hsem).wait()

      @plsc.parallel_loop(0, _VPAD, 1, unroll=8)
      def _zero(i):
        hist_v[pl.ds(i * _LANES, _LANES)] = zf

      pltpu.make_async_copy(bos_src(0), bos_v, bsem).wait()

      nb = (p + 1) % 2

      @pl.when(g + 1 < _GROUPS)
      def _prefetch():
        pltpu.async_copy(bos_src(g + 1), bos_bufs[nb], bos_sems[nb])

      @plsc.parallel_loop(0, _HIST, 1, unroll=8)
      def _scatter(l):
        tokc = plsc.load_gather(bos_v, [col0 + l])
        plsc.addupdate_scatter(hist_v, [tokc + rowoff], ones)

      pltpu.async_copy(hist_v, hist_dst(g), hsem)
    return 0

  lax.fori_loop(0, _GROUPS // 2, pair_body, 0)

  pltpu.make_async_copy(hist_a, hist_dst(0), sem_ha).wait()
  pltpu.make_async_copy(hist_b, hist_dst(0), sem_hb).wait()


def _mm_body(hist_ref, tab_ref, out_ref):
  h = hist_ref[...].reshape(_BM, _VPAD).astype(jnp.bfloat16)
  r = jnp.dot(h, tab_ref[...], preferred_element_type=jnp.float32)
  out_ref[...] = r[:, :_D] / r[:, _D:_D + 1]


@functools.partial(jax.jit, donate_argnums=())
def _run(bos, tab_aug):
  mesh = plsc.VectorSubcoreMesh(core_axis_name="c", subcore_axis_name="s")
  hist_k = pl.kernel(
      _hist_body,
      out_type=jax.ShapeDtypeStruct((_CB * _VPAD,), jnp.float32),
      mesh=mesh,
      scratch_types=[
          pltpu.VMEM((_GBOS,), jnp.int32),
          pltpu.VMEM((_GBOS,), jnp.int32),
          pltpu.VMEM((_GHIST,), jnp.float32),
          pltpu.VMEM((_GHIST,), jnp.float32),
          pltpu.SemaphoreType.DMA,
          pltpu.SemaphoreType.DMA,
          pltpu.SemaphoreType.DMA,
          pltpu.SemaphoreType.DMA,
      ],
      compiler_params=pltpu.CompilerParams(needs_layout_passes=False),
  )
  mm = pl.pallas_call(
      _mm_body,
      grid=(_CB // _BM,),
      in_specs=[
          pl.BlockSpec((_BM * _VPAD,), lambda i: (i,)),
          pl.BlockSpec((_VPAD, _N), lambda i: (0, 0)),
      ],
      out_specs=pl.BlockSpec((_BM, _D), lambda i: (i, 0)),
      out_shape=jax.ShapeDtypeStruct((_CB, _D), jnp.float32),
  )
  outs = []
  for i in range(_NCHB):
    flat_i = lax.slice_in_dim(bos, i * _CB, (i + 1) * _CB).reshape(-1)
    outs.append(mm(hist_k(flat_i), tab_aug))
  return jnp.concatenate(outs, 0)


def kernel(bos, table):
  tab_eff = table.at[0].set(0.0)
  ones_col = jnp.ones((_VOCAB, 1), jnp.float32).at[0, 0].set(0.0)
  pad = jnp.zeros((_VOCAB, _N - _D - 1), jnp.float32)
  tab_aug = jnp.concatenate([tab_eff, ones_col, pad], 1).astype(jnp.bfloat16)
  tab_aug = jnp.pad(tab_aug, ((0, _VPAD - _VOCAB), (0, 0)))
  return _run(bos, tab_aug)
